# Initial kernel scaffold; baseline (speedup 1.0000x reference)
import functools

import jax
import jax.numpy as jnp
from jax import lax
from jax.experimental import pallas as pl
from jax.experimental.pallas import tpu as pltpu
from jax.experimental.pallas import tpu_sc as plsc

N = 10000
NE = 10000
D = 128
E = 320000
C = 128
NCHUNK = E // C

NC = 2
NS = 16
NW = NC * NS


def _seg_accumulate(nseg, with_counts):
  rows_per_tile = nseg // NS
  zrows = rows_per_tile // 5

  def body(*refs):
    if with_counts:
      (src_hbm, gidx_hbm, sidx_hbm, out_hbm, cnt_hbm,
       gidx_v, sidx_v, rows_v, zbuf_v, zbuf16_v, ones_v, acc_sh, cnt_sh,
       sem) = refs
    else:
      (src_hbm, gidx_hbm, sidx_hbm, out_hbm,
       gidx_v, sidx_v, rows_v, zbuf_v, acc_sh, sem) = refs

    cid = lax.axis_index("c")
    sid = lax.axis_index("s")
    w = sid * NC + cid

    zf = jnp.zeros((16,), jnp.float32)

    def zfill(r, carry):
      for k in range(8):
        zbuf_v[r, pl.ds(16 * k, 16)] = zf
      return carry
    lax.fori_loop(0, zrows, zfill, 0)

    if with_counts:
      of = jnp.ones((16,), jnp.float32)
      def zfill16(r, carry):
        zbuf16_v[r, :] = zf
        return carry
      lax.fori_loop(0, zrows, zfill16, 0)
      def ofill(r, carry):
        ones_v[r, :] = of
        return carry
      lax.fori_loop(0, C, ofill, 0)

    base = sid * rows_per_tile
    for k in range(5):
      pltpu.sync_copy(zbuf_v, acc_sh.at[pl.ds(base + k * zrows, zrows)])
      if with_counts:
        pltpu.sync_copy(zbuf16_v, cnt_sh.at[pl.ds(base + k * zrows, zrows)])
    plsc.subcore_barrier()

    n_w = 78 + jnp.where(w < NCHUNK - 78 * NW, 1, 0)

    def step(j, carry):
      g = w + j * NW
      pltpu.sync_copy(gidx_hbm.at[g], gidx_v)
      pltpu.sync_copy(sidx_hbm.at[g], sidx_v)
      pltpu.async_copy(src_hbm.at[gidx_v], rows_v, sem).wait()
      pltpu.sync_copy(rows_v, acc_sh.at[sidx_v], add=True)
      if with_counts:
        pltpu.sync_copy(ones_v, cnt_sh.at[sidx_v], add=True)
      return carry
    lax.fori_loop(0, n_w, step, 0)

    plsc.subcore_barrier()

    pltpu.sync_copy(acc_sh.at[pl.ds(base, rows_per_tile)],
                    out_hbm.at[cid, pl.ds(base, rows_per_tile)])
    if with_counts:
      pltpu.sync_copy(cnt_sh.at[pl.ds(base, rows_per_tile)],
                      cnt_hbm.at[cid, pl.ds(base, rows_per_tile)])

  out_type = [jax.ShapeDtypeStruct((NC, nseg, D), jnp.float32)]
  scratch = [
      pltpu.VMEM((C,), jnp.int32),
      pltpu.VMEM((C,), jnp.int32),
      pltpu.VMEM((C, D), jnp.float32),
      pltpu.VMEM((zrows, D), jnp.float32),
  ]
  if with_counts:
    out_type.append(jax.ShapeDtypeStruct((NC, nseg, 16), jnp.float32))
    scratch += [
        pltpu.VMEM((zrows, 16), jnp.float32),
        pltpu.VMEM((C, 16), jnp.float32),
    ]
  scratch.append(pltpu.VMEM_SHARED((nseg, D), jnp.float32))
  if with_counts:
    scratch.append(pltpu.VMEM_SHARED((nseg, 16), jnp.float32))
  scratch.append(pltpu.SemaphoreType.DMA)

  mesh = plsc.VectorSubcoreMesh(core_axis_name="c", subcore_axis_name="s")
  return pl.kernel(body, out_type=tuple(out_type), mesh=mesh,
                   scratch_types=tuple(scratch))


_seg_sum_counts = _seg_accumulate(NE, True)
_seg_sum_e = _seg_accumulate(NE, False)
_seg_sum_v = _seg_accumulate(N, False)

BLK = 500


def _mm_body(x_ref, wt_ref, o_ref):
  o_ref[...] = jnp.dot(x_ref[...], wt_ref[...],
                       preferred_element_type=jnp.float32)


_matmul = pl.pallas_call(
    _mm_body,
    grid=(N // BLK,),
    in_specs=[pl.BlockSpec((BLK, D), lambda i: (i, 0)),
              pl.BlockSpec((D, D), lambda i: (0, 0))],
    out_specs=pl.BlockSpec((BLK, D), lambda i: (i, 0)),
    out_shape=jax.ShapeDtypeStruct((N, D), jnp.float32),
)


def _fused_mm_body(s_ref, x_ref, q0_ref, q1_ref, wt_ref, o_ref):
  h = jax.nn.relu(s_ref[0, 0] * x_ref[...] + q0_ref[...] + q1_ref[...])
  o_ref[...] = jnp.dot(h, wt_ref[...], preferred_element_type=jnp.float32)


_fused_matmul = pl.pallas_call(
    _fused_mm_body,
    grid=(N // BLK,),
    in_specs=[pl.BlockSpec(memory_space=pltpu.SMEM),
              pl.BlockSpec((BLK, D), lambda i: (i, 0)),
              pl.BlockSpec((BLK, D), lambda i: (i, 0)),
              pl.BlockSpec((BLK, D), lambda i: (i, 0)),
              pl.BlockSpec((D, D), lambda i: (0, 0))],
    out_specs=pl.BlockSpec((BLK, D), lambda i: (i, 0)),
    out_shape=jax.ShapeDtypeStruct((N, D), jnp.float32),
)


def _mean_body(p0_ref, p1_ref, c0_ref, c1_ref, o_ref):
  c = c0_ref[...] + c1_ref[...]
  r = 1.0 / jnp.maximum(c[:, :1], 1.0)
  o_ref[...] = (p0_ref[...] + p1_ref[...]) * r


_seg_mean = pl.pallas_call(
    _mean_body,
    grid=(NE // BLK,),
    in_specs=[pl.BlockSpec((BLK, D), lambda i: (i, 0)),
              pl.BlockSpec((BLK, D), lambda i: (i, 0)),
              pl.BlockSpec((BLK, 16), lambda i: (i, 0)),
              pl.BlockSpec((BLK, 16), lambda i: (i, 0))],
    out_specs=pl.BlockSpec((BLK, D), lambda i: (i, 0)),
    out_shape=jax.ShapeDtypeStruct((NE, D), jnp.float32),
)


def _final_body(s_ref, x_ref, q0_ref, q1_ref, o_ref):
  o_ref[...] = s_ref[0, 0] * x_ref[...] + q0_ref[...] + q1_ref[...]


_final_combine = pl.pallas_call(
    _final_body,
    grid=(N // BLK,),
    in_specs=[pl.BlockSpec(memory_space=pltpu.SMEM),
              pl.BlockSpec((BLK, D), lambda i: (i, 0)),
              pl.BlockSpec((BLK, D), lambda i: (i, 0)),
              pl.BlockSpec((BLK, D), lambda i: (i, 0))],
    out_specs=pl.BlockSpec((BLK, D), lambda i: (i, 0)),
    out_shape=jax.ShapeDtypeStruct((N, D), jnp.float32),
)


@jax.jit
def kernel(x, hyperedge_index, W1, eps1, W2, eps2):
  vertex2 = hyperedge_index[0].reshape(NCHUNK, C)
  edges2 = hyperedge_index[1].reshape(NCHUNK, C)
  s1 = (1.0 + eps1).reshape(1, 1)
  s2 = (1.0 + eps2).reshape(1, 1)
  W1t = W1.T
  W2t = W2.T

  X1 = _matmul(x, W1t)
  P1, CNT = _seg_sum_counts(X1, vertex2, edges2)
  Xe1 = _seg_mean(P1[0], P1[1], CNT[0], CNT[1])
  Q1 = _seg_sum_v(Xe1, edges2, vertex2)
  X2 = _fused_matmul(s1, X1, Q1[0], Q1[1], W2t)
  P2 = _seg_sum_e(X2, vertex2, edges2)
  Xe2 = _seg_mean(P2[0], P2[1], CNT[0], CNT[1])
  Q2 = _seg_sum_v(Xe2, edges2, vertex2)
  Xout = _final_combine(s2, X2, Q2[0], Q2[1])
  return (Xout, Xe2)


# speedup vs baseline: 10.2309x; 10.2309x over previous
import jax
import jax.numpy as jnp
from jax import lax
from jax.experimental import pallas as pl
from jax.experimental.pallas import tpu as pltpu
from jax.experimental.pallas import tpu_sc as plsc

N = 10000
NE = 10000
D = 128
E = 320000
C = 128
NCHUNK = E // C

NC = 2
NS = 16
NW = NC * NS
NCH = 80
NLAST = 16
NTAIL = NCHUNK - 31 * NCH - NLAST
NB = 4
NPAD = 10240


def _seg_accumulate():
  rows_per_tile = NPAD // NS
  zrows = 16

  def body(src_hbm, gidx_hbm, sidx_hbm, out_hbm,
           gb0, gb1, sb0, sb1, rows_v, zbuf_v, acc_sh, isem, gsem, ssem):
    gbufs = (gb0, gb1)
    sbufs = (sb0, sb1)
    cid = lax.axis_index("c")
    sid = lax.axis_index("s")
    w = sid * NC + cid
    cbase = w * NCH
    n_w = jnp.where(w < NW - 1, NCH, NLAST)

    def fire_idx(j, b):
      pltpu.async_copy(gidx_hbm.at[cbase + j], gbufs[b], isem.at[b])
      pltpu.async_copy(sidx_hbm.at[cbase + j], sbufs[b], isem.at[b])

    def wait_idx(j, b):
      pltpu.make_async_copy(gidx_hbm.at[cbase + j], gbufs[b], isem.at[b]).wait()
      pltpu.make_async_copy(sidx_hbm.at[cbase + j], sbufs[b], isem.at[b]).wait()

    def fire_gather(b):
      pltpu.async_copy(src_hbm.at[gbufs[b]], rows_v.at[b], gsem.at[b])

    def wait_gather(b):
      pltpu.make_async_copy(src_hbm.at[gbufs[b]], rows_v.at[b],
                            gsem.at[b]).wait()

    def scatter(b):
      pltpu.async_copy(rows_v.at[b], acc_sh.at[sbufs[b]],
                       ssem.at[b], add=True).wait()

    fire_idx(0, 0)
    fire_idx(1, 1)
    wait_idx(0, 0)
    fire_gather(0)

    zf = jnp.zeros((16,), jnp.float32)
    def zfill(r, carry):
      for k in range(8):
        zbuf_v[r, pl.ds(16 * k, 16)] = zf
      return carry
    lax.fori_loop(0, zrows, zfill, 0)

    base = sid * rows_per_tile
    for k in range(rows_per_tile // zrows):
      pltpu.sync_copy(zbuf_v, acc_sh.at[pl.ds(base + k * zrows, zrows)])
    plsc.subcore_barrier()

    def full_iter(j, b, bn):
      wait_idx(j + 1, bn)
      fire_gather(bn)
      wait_gather(b)
      scatter(b)
      fire_idx(j + 2, b)

    def pair(jj, carry):
      j = jj * 2
      full_iter(j, 0, 1)
      full_iter(j + 1, 1, 0)
      return carry
    lax.fori_loop(0, (n_w - 2) // 2, pair, 0)
    wait_idx(n_w - 1, 1)
    fire_gather(1)
    wait_gather(0)
    scatter(0)
    wait_gather(1)
    scatter(1)

    @pl.when(w == NW - 1)
    def _tail():
      for t in range(NTAIL):
        g = 31 * NCH + NLAST + t
        pltpu.sync_copy(gidx_hbm.at[g], gb0)
        pltpu.sync_copy(sidx_hbm.at[g], sb0)
        pltpu.async_copy(src_hbm.at[gb0], rows_v.at[0], gsem.at[0]).wait()
        pltpu.async_copy(rows_v.at[0], acc_sh.at[sb0],
                         ssem.at[0], add=True).wait()

    plsc.subcore_barrier()

    pltpu.sync_copy(acc_sh.at[pl.ds(base, rows_per_tile)],
                    out_hbm.at[cid, pl.ds(base, rows_per_tile)])

  scratch = (
      pltpu.VMEM((C,), jnp.int32),
      pltpu.VMEM((C,), jnp.int32),
      pltpu.VMEM((C,), jnp.int32),
      pltpu.VMEM((C,), jnp.int32),
      pltpu.VMEM((2, C, D), jnp.float32),
      pltpu.VMEM((zrows, D), jnp.float32),
      pltpu.VMEM_SHARED((NPAD, D), jnp.float32),
      pltpu.SemaphoreType.DMA((2,)),
      pltpu.SemaphoreType.DMA((2,)),
      pltpu.SemaphoreType.DMA((2,)),
  )
  mesh = plsc.VectorSubcoreMesh(core_axis_name="c", subcore_axis_name="s",
                                num_cores=NC, num_subcores=NS)
  return pl.kernel(body, out_type=jax.ShapeDtypeStruct((NC, NPAD, D), jnp.float32),
                   mesh=mesh, scratch_types=scratch)


def _make_count_kernel():
  rows_per_tile = NPAD // NS
  zrows = 64

  def body(sidx_hbm, cnt_hbm, sidx_all,
           sb0, sb1, sb2, sb3,
           ones_v, zbuf_v, cnt_sh, ssem):
    sbufs = (sb0, sb1, sb2, sb3)
    cid = lax.axis_index("c")
    sid = lax.axis_index("s")
    w = sid * NC + cid
    cbase = pl.multiple_of(w * NCH, 8)
    n_w = jnp.where(w < NW - 1, NCH, NLAST)

    @pl.when(w < NW - 1)
    def _stage_full():
      pltpu.sync_copy(sidx_hbm.at[pl.ds(cbase, NCH)], sidx_all)

    @pl.when(w == NW - 1)
    def _stage_last():
      pltpu.sync_copy(sidx_hbm.at[pl.ds(cbase, NLAST)],
                      sidx_all.at[pl.ds(0, NLAST)])

    zf = jnp.zeros((16,), jnp.float32)
    of = jnp.ones((16,), jnp.float32)
    def fill(r, carry):
      for k in range(8):
        zbuf_v[r, pl.ds(16 * k, 16)] = zf
      return carry
    lax.fori_loop(0, zrows, fill, 0)
    def ofill(r, carry):
      for k in range(8):
        ones_v[r, pl.ds(16 * k, 16)] = of
      return carry
    lax.fori_loop(0, C, ofill, 0)

    base = sid * rows_per_tile
    for k in range(rows_per_tile // zrows):
      pltpu.sync_copy(zbuf_v, cnt_sh.at[pl.ds(base + k * zrows, zrows)])
    plsc.subcore_barrier()

    def load_sidx(j, b):
      for k in range(8):
        sbufs[b][pl.ds(16 * k, 16)] = sidx_all[j, pl.ds(16 * k, 16)]

    def fire(b):
      pltpu.async_copy(ones_v, cnt_sh.at[sbufs[b]], ssem.at[b], add=True)

    def drain(b):
      pltpu.make_async_copy(ones_v, cnt_sh.at[sbufs[b]], ssem.at[b]).wait()

    for b in range(NB):
      load_sidx(b, b)
      fire(b)
    def step(jj, carry):
      for b in range(NB):
        j = jj * NB + b
        drain(b)
        load_sidx(j + NB, b)
        fire(b)
      return carry
    lax.fori_loop(0, n_w // NB - 1, step, 0)
    for b in range(NB):
      drain(b)

    @pl.when(w == NW - 1)
    def _tail():
      for t in range(NTAIL):
        g = 31 * NCH + NLAST + t
        pltpu.sync_copy(sidx_hbm.at[g], sb0)
        pltpu.async_copy(ones_v, cnt_sh.at[sb0], ssem.at[0], add=True).wait()

    plsc.subcore_barrier()
    pltpu.sync_copy(cnt_sh.at[pl.ds(base, rows_per_tile)],
                    cnt_hbm.at[cid, pl.ds(base, rows_per_tile)])

  scratch = (
      pltpu.VMEM((NCH, C), jnp.int32),
      pltpu.VMEM((C,), jnp.int32),
      pltpu.VMEM((C,), jnp.int32),
      pltpu.VMEM((C,), jnp.int32),
      pltpu.VMEM((C,), jnp.int32),
      pltpu.VMEM((C, D), jnp.float32),
      pltpu.VMEM((64, D), jnp.float32),
      pltpu.VMEM_SHARED((NPAD, D), jnp.float32),
      pltpu.SemaphoreType.DMA((NB,)),
  )
  mesh = plsc.VectorSubcoreMesh(core_axis_name="c", subcore_axis_name="s",
                                num_cores=NC, num_subcores=NS)
  return pl.kernel(body, out_type=jax.ShapeDtypeStruct((NC, NPAD, D), jnp.float32),
                   mesh=mesh, scratch_types=scratch)


_seg_sum = _seg_accumulate()
_count_edges = _make_count_kernel()

BLK = 1000


def _mm_body(x_ref, wt_ref, o_ref):
  o_ref[...] = jnp.dot(x_ref[...], wt_ref[...],
                       preferred_element_type=jnp.float32)


_matmul = pl.pallas_call(
    _mm_body,
    grid=(N // BLK,),
    in_specs=[pl.BlockSpec((BLK, D), lambda i: (i, 0)),
              pl.BlockSpec((D, D), lambda i: (0, 0))],
    out_specs=pl.BlockSpec((BLK, D), lambda i: (i, 0)),
    out_shape=jax.ShapeDtypeStruct((N, D), jnp.float32),
)


def _fused_mm_body(s_ref, x_ref, q0_ref, q1_ref, wt_ref, o_ref):
  h = jax.nn.relu(s_ref[0, 0] * x_ref[...] + q0_ref[...] + q1_ref[...])
  o_ref[...] = jnp.dot(h, wt_ref[...], preferred_element_type=jnp.float32)


_fused_matmul = pl.pallas_call(
    _fused_mm_body,
    grid=(N // BLK,),
    in_specs=[pl.BlockSpec(memory_space=pltpu.SMEM),
              pl.BlockSpec((BLK, D), lambda i: (i, 0)),
              pl.BlockSpec((BLK, D), lambda i: (i, 0)),
              pl.BlockSpec((BLK, D), lambda i: (i, 0)),
              pl.BlockSpec((D, D), lambda i: (0, 0))],
    out_specs=pl.BlockSpec((BLK, D), lambda i: (i, 0)),
    out_shape=jax.ShapeDtypeStruct((N, D), jnp.float32),
)


def _mean_body(p0_ref, p1_ref, c0_ref, c1_ref, o_ref):
  c = c0_ref[...] + c1_ref[...]
  r = 1.0 / jnp.maximum(c[:, :1], 1.0)
  o_ref[...] = (p0_ref[...] + p1_ref[...]) * r


_seg_mean = pl.pallas_call(
    _mean_body,
    grid=(NE // BLK,),
    in_specs=[pl.BlockSpec((BLK, D), lambda i: (i, 0)),
              pl.BlockSpec((BLK, D), lambda i: (i, 0)),
              pl.BlockSpec((BLK, D), lambda i: (i, 0)),
              pl.BlockSpec((BLK, D), lambda i: (i, 0))],
    out_specs=pl.BlockSpec((BLK, D), lambda i: (i, 0)),
    out_shape=jax.ShapeDtypeStruct((NE, D), jnp.float32),
)


def _final_body(s_ref, x_ref, q0_ref, q1_ref, o_ref):
  o_ref[...] = s_ref[0, 0] * x_ref[...] + q0_ref[...] + q1_ref[...]


_final_combine = pl.pallas_call(
    _final_body,
    grid=(N // BLK,),
    in_specs=[pl.BlockSpec(memory_space=pltpu.SMEM),
              pl.BlockSpec((BLK, D), lambda i: (i, 0)),
              pl.BlockSpec((BLK, D), lambda i: (i, 0)),
              pl.BlockSpec((BLK, D), lambda i: (i, 0))],
    out_specs=pl.BlockSpec((BLK, D), lambda i: (i, 0)),
    out_shape=jax.ShapeDtypeStruct((N, D), jnp.float32),
)


@jax.jit
def kernel(x, hyperedge_index, W1, eps1, W2, eps2):
  vertex2 = hyperedge_index[0].reshape(NCHUNK, C)
  edges2 = hyperedge_index[1].reshape(NCHUNK, C)
  s1 = (1.0 + eps1).reshape(1, 1)
  s2 = (1.0 + eps2).reshape(1, 1)
  W1t = W1.T
  W2t = W2.T

  CNT = _count_edges(edges2)
  X1 = _matmul(x, W1t)
  P1 = _seg_sum(X1, vertex2, edges2)
  Xe1 = _seg_mean(P1[0, :NE], P1[1, :NE], CNT[0, :NE], CNT[1, :NE])
  Q1 = _seg_sum(Xe1, edges2, vertex2)
  X2 = _fused_matmul(s1, X1, Q1[0, :N], Q1[1, :N], W2t)
  P2 = _seg_sum(X2, vertex2, edges2)
  Xe2 = _seg_mean(P2[0, :NE], P2[1, :NE], CNT[0, :NE], CNT[1, :NE])
  Q2 = _seg_sum(Xe2, edges2, vertex2)
  Xout = _final_combine(s2, X2, Q2[0, :N], Q2[1, :N])
  return (Xout, Xe2)
